# baseline (speedup 1.0000x reference)
import jax
import jax.numpy as jnp
import numpy as np
from jax.experimental import pallas as pl
from jax.experimental.pallas import tpu as pltpu

N = 10000
BLK = 1024
NBLK = 10
NP = BLK * NBLK
D = 128
NBES = 8
HID = 8
NB_GRAPH = 64
DIST = 5.0
EPS = 1e-6
NEG = -1e30
SQRT_D = float(np.sqrt(D))

_R_CONST = np.repeat(np.eye(16, dtype=np.float32), NB_GRAPH, axis=1)

_HI = jax.lax.Precision.HIGHEST
_LO = jax.lax.Precision.DEFAULT


def _dot(a, b, dims, prec=_HI):
    return jax.lax.dot_general(a, b, (dims, ((), ())), precision=prec,
                               preferred_element_type=jnp.float32)


def _body(bounds_ref, z_ref, b_ref, x_ref, bw_ref, init_ref, Wqd_ref,
          Wdot_ref, Wqg_ref, Wk1_ref, Wv1_ref, Wkg1_ref, Wvg1_ref, Wk2_ref,
          Wv2_ref, Wkg2_ref, Wvg2_ref, R_ref, out_ref, S_ref, Ahat_ref,
          K2G_ref, V2g_ref, KE_ref, WM_ref, BF_ref, WQD_ref):
    ph = pl.program_id(0)
    j = pl.program_id(1)

    @pl.when((ph == 0) & (j == 0))
    def _init():
        ci = jax.lax.broadcasted_iota(jnp.int32, (NB_GRAPH, 8), 1)
        S_ref[...] = jnp.where(ci == 0, 1e30,
                               jnp.where(ci < 4, NEG, 0.0)).astype(jnp.float32)
        Ahat_ref[...] = jnp.zeros((NB_GRAPH, 16, D), jnp.float32)

    zrow = z_ref[...]
    brow = b_ref[...]
    iota = jax.lax.broadcasted_iota(jnp.int32, (NB_GRAPH, BLK), 0)
    Pb = brow == iota
    Pf = Pb.astype(jnp.float32)

    @pl.when(ph == 0)
    def _zstats():
        zmin = jnp.min(jnp.where(Pb, zrow, 1e30), axis=1, keepdims=True)
        zmax = jnp.max(jnp.where(Pb, zrow, NEG), axis=1, keepdims=True)
        S_ref[:, 0:1] = jnp.minimum(S_ref[:, 0:1], zmin)
        S_ref[:, 1:2] = jnp.maximum(S_ref[:, 1:2], zmax)

    def _mixed_feats(bf_t, wm):
        h = jax.nn.silu(_dot(wm, bf_t, ((0,), (0,))))
        return h[0:8], h[8:16], h[16:24], h[24:32]

    @pl.when((ph == 1) & (j == 0))
    def _shared_weights():
        q0 = _dot(init_ref[...], Wqd_ref[...], ((1,), (0,)))
        wv = _dot(q0, Wdot_ref[...], ((1,), (0,))) / SQRT_D
        KE_ref[...] = jnp.concatenate(
            [_dot(Wk2_ref[h], wv, ((1,), (1,))) for h in range(HID)], axis=1)
        z8 = jnp.zeros((8, 8), jnp.float32)
        blocks = []
        for W in (Wk1_ref[...], Wv1_ref[...], Wkg1_ref[...], Wvg1_ref[...]):
            blocks.append(jnp.concatenate([W, z8], 0))
            blocks.append(jnp.concatenate([z8, W], 0))
        WM_ref[...] = jnp.concatenate(blocks, 1)
        WQD_ref[...] = _dot(Wqg_ref[...], Wdot_ref[...], ((1,), (0,))) / SQRT_D

    @pl.when(ph == 1)
    def _main():
        X = x_ref[...]
        stats = _dot(S_ref[...], Pf, ((0,), (0,)))
        bottom = zrow - stats[0:1] + DIST
        top = stats[1:2] + DIST - zrow
        bw = bw_ref[...]
        bfin = jnp.concatenate([bw * bottom, bw * top], 0)
        denv = jnp.concatenate(
            [jnp.broadcast_to(bottom + EPS, (NBES, BLK)),
             jnp.broadcast_to(top + EPS, (NBES, BLK))], 0)
        bf_t = jnp.sin(bfin) / denv
        BF_ref[j] = bf_t
        hk0, hk1, hv0, hv1 = _mixed_feats(bf_t, WM_ref[:, 0:32])
        s_k = _dot(KE_ref[...], X, ((0,), (1,)))
        l0 = jnp.sum(hk0 * s_k, axis=0, keepdims=True)
        l1 = jnp.sum(hk1 * s_k, axis=0, keepdims=True)
        bm0 = jnp.max(jnp.where(Pb, l0, NEG), axis=1, keepdims=True)
        bm1 = jnp.max(jnp.where(Pb, l1, NEG), axis=1, keepdims=True)
        old0 = S_ref[:, 2:3]
        old1 = S_ref[:, 3:4]
        new0 = jnp.maximum(old0, bm0)
        new1 = jnp.maximum(old1, bm1)
        sc0 = jnp.exp(old0 - new0)
        sc1 = jnp.exp(old1 - new1)
        S_ref[:, 2:3] = new0
        S_ref[:, 3:4] = new1
        m_n = _dot(jnp.concatenate([new0, new1], 1), Pf, ((0,), (0,)))
        e0 = jnp.exp(l0 - m_n[0:1])
        e1 = jnp.exp(l1 - m_n[1:2])
        sum_e = _dot(Pf, jnp.concatenate([e0, e1], 0), ((1,), (1,)))
        S_ref[:, 4:6] = (S_ref[:, 4:6] * jnp.concatenate([sc0, sc1], 1)
                         + sum_e)
        coeff = jnp.concatenate([e0 * hv0, e1 * hv1], 0)
        s64 = jnp.concatenate([jnp.broadcast_to(sc0, (NB_GRAPH, NBES)),
                               jnp.broadcast_to(sc1, (NB_GRAPH, NBES))], 1)
        Ahat_ref[...] = Ahat_ref[...] * s64[:, :, None]
        lo = bounds_ref[j, 0]
        hi = bounds_ref[j, 1]

        def _acc(b, _):
            maskf = (brow == b).astype(jnp.float32)
            delta = _dot(coeff * maskf, X, ((1,), (0,)), _LO)
            Ahat_ref[b] = Ahat_ref[b] + delta
            return 0

        jax.lax.fori_loop(lo, hi + 1, _acc, 0)

    @pl.when((ph == 2) & (j == 0))
    def _per_graph():
        A3 = Ahat_ref[...]
        for p in range(2):
            du = jnp.zeros((NB_GRAPH, D), jnp.float32)
            for h in range(HID):
                du = du + _dot(A3[:, p * HID + h, :], Wv2_ref[h], ((1,), (0,)))
            inv_col = 1.0 / (S_ref[:, 4 + p:5 + p] + 1e-9)
            nd = jnp.broadcast_to(init_ref[...], (NB_GRAPH, D)) + du * inv_col
            for h in range(HID):
                k2g = _dot(nd, Wkg2_ref[h], ((1,), (0,)))
                K2G_ref[:, p * HID + h, :] = _dot(
                    k2g, WQD_ref[...], ((1,), (1,)))
                V2g_ref[p * HID + h] = _dot(nd, Wvg2_ref[h], ((1,), (0,)))

    @pl.when(ph == 2)
    def _out():
        X = x_ref[...]
        bf_t = BF_ref[j]
        hkg0, hkg1, hvg0, hvg1 = _mixed_feats(bf_t, WM_ref[:, 32:64])
        lo = bounds_ref[j, 0]
        hi = bounds_ref[j, 1]

        def _gather_t(b, t):
            maskf = (brow == b).astype(jnp.float32)
            tb = _dot(K2G_ref[b], X, ((1,), (1,)), _LO)
            return t + tb * maskf

        T = jax.lax.fori_loop(lo, hi + 1, _gather_t,
                              jnp.zeros((16, BLK), jnp.float32))
        lg0 = jnp.sum(hkg0 * T[0:HID], axis=0, keepdims=True)
        lg1 = jnp.sum(hkg1 * T[HID:16], axis=0, keepdims=True)
        mx = jnp.maximum(lg0, lg1)
        a0 = jnp.exp(lg0 - mx)
        a1 = jnp.exp(lg1 - mx)
        s = a0 + a1
        coeff_g = jnp.concatenate([(a0 / s) * hvg0, (a1 / s) * hvg1], 0)
        Ptile = jnp.concatenate([Pf] * 16, axis=0)
        coeffB = _dot(R_ref[...], coeff_g, ((0,), (0,)), _LO) * Ptile
        upd = _dot(coeffB, V2g_ref[...].reshape(16 * NB_GRAPH, D),
                   ((0,), (0,)), _LO)
        out_ref[...] = X + upd


def kernel(pos, node_features, batch, bessel_weights, init_dummy_embedding,
           Wq_dummy, Wk1, Wk2, Wv1, Wv2, Wq_graph, Wkg1, Wkg2, Wvg1, Wvg2,
           Wdot):
    pad = NP - N
    zrow = jnp.pad(pos[:, 2], (0, pad)).reshape(1, NP)
    brow = jnp.pad(batch, (0, pad), constant_values=NB_GRAPH).reshape(1, NP)
    xpad = jnp.pad(node_features, ((0, pad), (0, 0)))
    bwc = bessel_weights.reshape(NBES, 1)
    init2 = init_dummy_embedding.reshape(1, D)
    R = jnp.asarray(_R_CONST)
    starts = jnp.arange(NBLK, dtype=jnp.int32) * BLK
    lo = batch[jnp.minimum(starts, N - 1)].astype(jnp.int32)
    hi = batch[jnp.minimum(starts + BLK - 1, N - 1)].astype(jnp.int32)
    bounds = jnp.stack([lo, hi], axis=1)

    row_map = lambda ph, j: (0, j)
    x_map = lambda ph, j: (jnp.where(ph == 0, 0, j), 0)
    out_map = lambda ph, j: (jnp.where(ph == 2, j, 0), 0)
    full2 = lambda ph, j: (0, 0)
    full3 = lambda ph, j: (0, 0, 0)

    in_specs = [
        pl.BlockSpec(memory_space=pltpu.SMEM),
        pl.BlockSpec((1, BLK), row_map),
        pl.BlockSpec((1, BLK), row_map),
        pl.BlockSpec((BLK, D), x_map),
        pl.BlockSpec((NBES, 1), full2),
        pl.BlockSpec((1, D), full2),
        pl.BlockSpec((D, D), full2),
        pl.BlockSpec((D, D), full2),
        pl.BlockSpec((D, D), full2),
        pl.BlockSpec((NBES, HID), full2),
        pl.BlockSpec((NBES, HID), full2),
        pl.BlockSpec((NBES, HID), full2),
        pl.BlockSpec((NBES, HID), full2),
        pl.BlockSpec((HID, D, D), full3),
        pl.BlockSpec((HID, D, D), full3),
        pl.BlockSpec((HID, D, D), full3),
        pl.BlockSpec((HID, D, D), full3),
        pl.BlockSpec((16, 16 * NB_GRAPH), full2),
    ]

    out = pl.pallas_call(
        _body,
        grid=(3, NBLK),
        in_specs=in_specs,
        out_specs=pl.BlockSpec((BLK, D), out_map),
        out_shape=jax.ShapeDtypeStruct((NP, D), jnp.float32),
        scratch_shapes=[
            pltpu.VMEM((NB_GRAPH, 8), jnp.float32),
            pltpu.VMEM((NB_GRAPH, 16, D), jnp.float32),
            pltpu.VMEM((NB_GRAPH, 16, D), jnp.float32),
            pltpu.VMEM((16, NB_GRAPH, D), jnp.float32),
            pltpu.VMEM((D, HID), jnp.float32),
            pltpu.VMEM((16, 64), jnp.float32),
            pltpu.VMEM((NBLK, 16, BLK), jnp.float32),
            pltpu.VMEM((D, D), jnp.float32),
        ],
    )(bounds, zrow, brow, xpad, bwc, init2, Wq_dummy, Wdot, Wq_graph,
      Wk1, Wv1, Wkg1, Wvg1, Wk2, Wv2, Wkg2, Wvg2, R)
    return out[:N]


# speedup vs baseline: 1.2808x; 1.2808x over previous
import jax
import jax.numpy as jnp
import numpy as np
from jax.experimental import pallas as pl
from jax.experimental.pallas import tpu as pltpu

N = 10000
BLK = 2048
NBLK = 5
NP = BLK * NBLK
D = 128
NBES = 8
HID = 8
NB_GRAPH = 64
DIST = 5.0
EPS = 1e-6
NEG = -1e30
SQRT_D = float(np.sqrt(D))

_R_CONST = np.repeat(np.eye(16, dtype=np.float32), NB_GRAPH, axis=1)

_HI = jax.lax.Precision.HIGHEST
_LO = jax.lax.Precision.DEFAULT


def _dot(a, b, dims, prec=_HI):
    return jax.lax.dot_general(a, b, (dims, ((), ())), precision=prec,
                               preferred_element_type=jnp.float32)


def _body(z_ref, b_ref, x_ref, bw_ref, init_ref, Wqd_ref, Wdot_ref, Wqg_ref,
          Wk1_ref, Wv1_ref, Wkg1_ref, Wvg1_ref, Wk2_ref, Wv2_ref, Wkg2_ref,
          Wvg2_ref, R_ref, out_ref, S_ref, Ahat_ref, Mq_ref, V2g_ref, KE_ref,
          WM_ref, BF_ref):
    ph = pl.program_id(0)
    j = pl.program_id(1)

    @pl.when((ph == 0) & (j == 0))
    def _init():
        ci = jax.lax.broadcasted_iota(jnp.int32, (NB_GRAPH, 8), 1)
        S_ref[...] = jnp.where(ci == 0, 1e30,
                               jnp.where(ci < 4, NEG, 0.0)).astype(jnp.float32)
        Ahat_ref[...] = jnp.zeros((16, NB_GRAPH, D), jnp.float32)

    zrow = z_ref[...]
    brow = b_ref[...]
    iota = jax.lax.broadcasted_iota(jnp.int32, (NB_GRAPH, BLK), 0)
    Pb = brow == iota
    Pf = Pb.astype(jnp.float32)

    @pl.when(ph == 0)
    def _zstats():
        zmin = jnp.min(jnp.where(Pb, zrow, 1e30), axis=1, keepdims=True)
        zmax = jnp.max(jnp.where(Pb, zrow, NEG), axis=1, keepdims=True)
        S_ref[:, 0:1] = jnp.minimum(S_ref[:, 0:1], zmin)
        S_ref[:, 1:2] = jnp.maximum(S_ref[:, 1:2], zmax)

    def _mixed_feats(bf_t, wm):
        h = jax.nn.silu(_dot(wm, bf_t, ((0,), (0,))))
        return h[0:8], h[8:16], h[16:24], h[24:32]

    @pl.when((ph == 1) & (j == 0))
    def _shared_weights():
        q0 = _dot(init_ref[...], Wqd_ref[...], ((1,), (0,)))
        wv = _dot(q0, Wdot_ref[...], ((1,), (0,))) / SQRT_D
        KE_ref[...] = jnp.concatenate(
            [_dot(Wk2_ref[h], wv, ((1,), (1,))) for h in range(HID)], axis=1)
        z8 = jnp.zeros((8, 8), jnp.float32)
        blocks = []
        for W in (Wk1_ref[...], Wv1_ref[...], Wkg1_ref[...], Wvg1_ref[...]):
            blocks.append(jnp.concatenate([W, z8], 0))
            blocks.append(jnp.concatenate([z8, W], 0))
        WM_ref[...] = jnp.concatenate(blocks, 1)

    @pl.when(ph == 1)
    def _main():
        X = x_ref[...]
        stats = _dot(S_ref[...], Pf, ((0,), (0,)))
        bottom = zrow - stats[0:1] + DIST
        top = stats[1:2] + DIST - zrow
        bw = bw_ref[...]
        bfin = jnp.concatenate([bw * bottom, bw * top], 0)
        denv = jnp.concatenate(
            [jnp.broadcast_to(bottom + EPS, (NBES, BLK)),
             jnp.broadcast_to(top + EPS, (NBES, BLK))], 0)
        bf_t = jnp.sin(bfin) / denv
        BF_ref[j] = bf_t
        hk0, hk1, hv0, hv1 = _mixed_feats(bf_t, WM_ref[:, 0:32])
        s_k = _dot(KE_ref[...], X, ((0,), (1,)))
        l0 = jnp.sum(hk0 * s_k, axis=0, keepdims=True)
        l1 = jnp.sum(hk1 * s_k, axis=0, keepdims=True)
        bm0 = jnp.max(jnp.where(Pb, l0, NEG), axis=1, keepdims=True)
        bm1 = jnp.max(jnp.where(Pb, l1, NEG), axis=1, keepdims=True)
        old0 = S_ref[:, 2:3]
        old1 = S_ref[:, 3:4]
        new0 = jnp.maximum(old0, bm0)
        new1 = jnp.maximum(old1, bm1)
        sc0 = jnp.exp(old0 - new0)
        sc1 = jnp.exp(old1 - new1)
        S_ref[:, 2:3] = new0
        S_ref[:, 3:4] = new1
        m_n = _dot(jnp.concatenate([new0, new1], 1), Pf, ((0,), (0,)))
        e0 = jnp.exp(l0 - m_n[0:1])
        e1 = jnp.exp(l1 - m_n[1:2])
        sum_e = _dot(Pf, jnp.concatenate([e0, e1], 0), ((1,), (1,)))
        S_ref[:, 4:6] = (S_ref[:, 4:6] * jnp.concatenate([sc0, sc1], 1)
                         + sum_e)
        coeff = jnp.concatenate([e0 * hv0, e1 * hv1], 0)
        coeff_rep = _dot(R_ref[...], coeff, ((0,), (0,)), _LO)
        Ptile = jnp.concatenate([Pf] * 16, axis=0)
        contrib = _dot(coeff_rep * Ptile, X, ((1,), (0,)), _LO)
        eye = (jax.lax.broadcasted_iota(jnp.int32, (NB_GRAPH, NB_GRAPH), 0)
               == jax.lax.broadcasted_iota(jnp.int32, (NB_GRAPH, NB_GRAPH), 1)
               ).astype(jnp.float32)
        sc0r = _dot(sc0, eye, ((0,), (0,)))
        sc1r = _dot(sc1, eye, ((0,), (0,)))
        s01 = jnp.concatenate([jnp.broadcast_to(sc0r, (8, NB_GRAPH)),
                               jnp.broadcast_to(sc1r, (8, NB_GRAPH))], 0)
        Ahat_ref[...] = (Ahat_ref[...] * s01[:, :, None]
                         + contrib.reshape(16, NB_GRAPH, D))

    @pl.when((ph == 2) & (j == 0))
    def _per_graph():
        A3 = Ahat_ref[...]
        Wqd_g = _dot(Wqg_ref[...], Wdot_ref[...], ((1,), (0,))) / SQRT_D
        mq_blocks = []
        for p in range(2):
            du = jnp.zeros((NB_GRAPH, D), jnp.float32)
            for h in range(HID):
                du = du + _dot(A3[p * HID + h], Wv2_ref[h], ((1,), (0,)))
            inv_col = 1.0 / (S_ref[:, 4 + p:5 + p] + 1e-9)
            nd = jnp.broadcast_to(init_ref[...], (NB_GRAPH, D)) + du * inv_col
            for h in range(HID):
                k2g = _dot(nd, Wkg2_ref[h], ((1,), (0,)))
                mq_blocks.append(_dot(Wqd_g, k2g, ((1,), (1,))))
                V2g_ref[p * HID + h] = _dot(nd, Wvg2_ref[h], ((1,), (0,)))
        Mq_ref[...] = jnp.concatenate(mq_blocks, axis=1)

    @pl.when(ph == 2)
    def _out():
        X = x_ref[...]
        bf_t = BF_ref[j]
        hkg0, hkg1, hvg0, hvg1 = _mixed_feats(bf_t, WM_ref[:, 32:64])
        Ptile = jnp.concatenate([Pf] * 16, axis=0)
        Dm = _dot(Mq_ref[...], X, ((0,), (1,)), _LO)
        T = _dot(R_ref[...], Dm * Ptile, ((1,), (0,)), _LO)
        lg0 = jnp.sum(hkg0 * T[0:HID], axis=0, keepdims=True)
        lg1 = jnp.sum(hkg1 * T[HID:16], axis=0, keepdims=True)
        mx = jnp.maximum(lg0, lg1)
        a0 = jnp.exp(lg0 - mx)
        a1 = jnp.exp(lg1 - mx)
        s = a0 + a1
        coeff_g = jnp.concatenate([(a0 / s) * hvg0, (a1 / s) * hvg1], 0)
        coeffB = _dot(R_ref[...], coeff_g, ((0,), (0,)), _LO) * Ptile
        upd = _dot(coeffB, V2g_ref[...].reshape(16 * NB_GRAPH, D),
                   ((0,), (0,)), _LO)
        out_ref[...] = X + upd


def kernel(pos, node_features, batch, bessel_weights, init_dummy_embedding,
           Wq_dummy, Wk1, Wk2, Wv1, Wv2, Wq_graph, Wkg1, Wkg2, Wvg1, Wvg2,
           Wdot):
    pad = NP - N
    zrow = jnp.pad(pos[:, 2], (0, pad)).reshape(1, NP)
    brow = jnp.pad(batch, (0, pad), constant_values=NB_GRAPH).reshape(1, NP)
    xpad = jnp.pad(node_features, ((0, pad), (0, 0)))
    bwc = bessel_weights.reshape(NBES, 1)
    init2 = init_dummy_embedding.reshape(1, D)
    R = jnp.asarray(_R_CONST)

    row_map = lambda ph, j: (0, j)
    x_map = lambda ph, j: (jnp.where(ph == 0, 0, j), 0)
    out_map = lambda ph, j: (jnp.where(ph == 2, j, 0), 0)
    full2 = lambda ph, j: (0, 0)
    full3 = lambda ph, j: (0, 0, 0)

    in_specs = [
        pl.BlockSpec((1, BLK), row_map),
        pl.BlockSpec((1, BLK), row_map),
        pl.BlockSpec((BLK, D), x_map),
        pl.BlockSpec((NBES, 1), full2),
        pl.BlockSpec((1, D), full2),
        pl.BlockSpec((D, D), full2),
        pl.BlockSpec((D, D), full2),
        pl.BlockSpec((D, D), full2),
        pl.BlockSpec((NBES, HID), full2),
        pl.BlockSpec((NBES, HID), full2),
        pl.BlockSpec((NBES, HID), full2),
        pl.BlockSpec((NBES, HID), full2),
        pl.BlockSpec((HID, D, D), full3),
        pl.BlockSpec((HID, D, D), full3),
        pl.BlockSpec((HID, D, D), full3),
        pl.BlockSpec((HID, D, D), full3),
        pl.BlockSpec((16, 16 * NB_GRAPH), full2),
    ]

    out = pl.pallas_call(
        _body,
        grid=(3, NBLK),
        in_specs=in_specs,
        out_specs=pl.BlockSpec((BLK, D), out_map),
        out_shape=jax.ShapeDtypeStruct((NP, D), jnp.float32),
        scratch_shapes=[
            pltpu.VMEM((NB_GRAPH, 8), jnp.float32),
            pltpu.VMEM((16, NB_GRAPH, D), jnp.float32),
            pltpu.VMEM((D, 16 * NB_GRAPH), jnp.float32),
            pltpu.VMEM((16, NB_GRAPH, D), jnp.float32),
            pltpu.VMEM((D, HID), jnp.float32),
            pltpu.VMEM((16, 64), jnp.float32),
            pltpu.VMEM((NBLK, 16, BLK), jnp.float32),
        ],
    )(zrow, brow, xpad, bwc, init2, Wq_dummy, Wdot, Wq_graph,
      Wk1, Wv1, Wkg1, Wvg1, Wk2, Wv2, Wkg2, Wvg2, R)
    return out[:N]


# speedup vs baseline: 1.2929x; 1.0095x over previous
import jax
import jax.numpy as jnp
import numpy as np
from jax.experimental import pallas as pl
from jax.experimental.pallas import tpu as pltpu

N = 10000
BLK = 2560
NBLK = 4
NP = BLK * NBLK
D = 128
NBES = 8
HID = 8
NB_GRAPH = 64
DIST = 5.0
EPS = 1e-6
NEG = -1e30
SQRT_D = float(np.sqrt(D))

_R_CONST = np.repeat(np.eye(16, dtype=np.float32), NB_GRAPH, axis=1)

_HI = jax.lax.Precision.HIGHEST
_LO = jax.lax.Precision.DEFAULT


def _dot(a, b, dims, prec=_HI):
    return jax.lax.dot_general(a, b, (dims, ((), ())), precision=prec,
                               preferred_element_type=jnp.float32)


def _body(z_ref, b_ref, x_ref, bw_ref, init_ref, Wqd_ref, Wdot_ref, Wqg_ref,
          Wk1_ref, Wv1_ref, Wkg1_ref, Wvg1_ref, Wk2_ref, Wv2_ref, Wkg2_ref,
          Wvg2_ref, R_ref, out_ref, S_ref, Ahat_ref, Mq_ref, V2g_ref, KE_ref,
          WM_ref, BF_ref):
    ph = pl.program_id(0)
    j = pl.program_id(1)

    @pl.when((ph == 0) & (j == 0))
    def _init():
        ci = jax.lax.broadcasted_iota(jnp.int32, (NB_GRAPH, 8), 1)
        S_ref[...] = jnp.where(ci == 0, 1e30,
                               jnp.where(ci < 4, NEG, 0.0)).astype(jnp.float32)
        Ahat_ref[...] = jnp.zeros((16, NB_GRAPH, D), jnp.float32)

    zrow = z_ref[...]
    brow = b_ref[...]
    iota = jax.lax.broadcasted_iota(jnp.int32, (NB_GRAPH, BLK), 0)
    Pb = brow == iota
    Pf = Pb.astype(jnp.float32)

    @pl.when(ph == 0)
    def _zstats():
        zmin = jnp.min(jnp.where(Pb, zrow, 1e30), axis=1, keepdims=True)
        zmax = jnp.max(jnp.where(Pb, zrow, NEG), axis=1, keepdims=True)
        S_ref[:, 0:1] = jnp.minimum(S_ref[:, 0:1], zmin)
        S_ref[:, 1:2] = jnp.maximum(S_ref[:, 1:2], zmax)

    def _mixed_feats(bf_t, wm):
        h = jax.nn.silu(_dot(wm, bf_t, ((0,), (0,))))
        return h[0:8], h[8:16], h[16:24], h[24:32]

    @pl.when((ph == 1) & (j == 0))
    def _shared_weights():
        q0 = _dot(init_ref[...], Wqd_ref[...], ((1,), (0,)))
        wv = _dot(q0, Wdot_ref[...], ((1,), (0,))) / SQRT_D
        KE_ref[...] = jnp.concatenate(
            [_dot(Wk2_ref[h], wv, ((1,), (1,))) for h in range(HID)], axis=1)
        z8 = jnp.zeros((8, 8), jnp.float32)
        blocks = []
        for W in (Wk1_ref[...], Wv1_ref[...], Wkg1_ref[...], Wvg1_ref[...]):
            blocks.append(jnp.concatenate([W, z8], 0))
            blocks.append(jnp.concatenate([z8, W], 0))
        WM_ref[...] = jnp.concatenate(blocks, 1)

    @pl.when(ph == 1)
    def _main():
        X = x_ref[...]
        stats = _dot(S_ref[...], Pf, ((0,), (0,)))
        bottom = zrow - stats[0:1] + DIST
        top = stats[1:2] + DIST - zrow
        bw = bw_ref[...]
        bfin = jnp.concatenate([bw * bottom, bw * top], 0)
        denv = jnp.concatenate(
            [jnp.broadcast_to(bottom + EPS, (NBES, BLK)),
             jnp.broadcast_to(top + EPS, (NBES, BLK))], 0)
        bf_t = jnp.sin(bfin) / denv
        BF_ref[j] = bf_t
        hk0, hk1, hv0, hv1 = _mixed_feats(bf_t, WM_ref[:, 0:32])
        s_k = _dot(KE_ref[...], X, ((0,), (1,)))
        l0 = jnp.sum(hk0 * s_k, axis=0, keepdims=True)
        l1 = jnp.sum(hk1 * s_k, axis=0, keepdims=True)
        bm0 = jnp.max(jnp.where(Pb, l0, NEG), axis=1, keepdims=True)
        bm1 = jnp.max(jnp.where(Pb, l1, NEG), axis=1, keepdims=True)
        old0 = S_ref[:, 2:3]
        old1 = S_ref[:, 3:4]
        new0 = jnp.maximum(old0, bm0)
        new1 = jnp.maximum(old1, bm1)
        sc0 = jnp.exp(old0 - new0)
        sc1 = jnp.exp(old1 - new1)
        S_ref[:, 2:3] = new0
        S_ref[:, 3:4] = new1
        m_n = _dot(jnp.concatenate([new0, new1], 1), Pf, ((0,), (0,)))
        e0 = jnp.exp(l0 - m_n[0:1])
        e1 = jnp.exp(l1 - m_n[1:2])
        sum_e = _dot(Pf, jnp.concatenate([e0, e1], 0), ((1,), (1,)))
        S_ref[:, 4:6] = (S_ref[:, 4:6] * jnp.concatenate([sc0, sc1], 1)
                         + sum_e)
        coeff = jnp.concatenate([e0 * hv0, e1 * hv1], 0)
        coeff_rep = _dot(R_ref[...], coeff, ((0,), (0,)), _LO)
        Ptile = jnp.concatenate([Pf] * 16, axis=0)
        contrib = _dot(coeff_rep * Ptile, X, ((1,), (0,)), _LO)
        eye = (jax.lax.broadcasted_iota(jnp.int32, (NB_GRAPH, NB_GRAPH), 0)
               == jax.lax.broadcasted_iota(jnp.int32, (NB_GRAPH, NB_GRAPH), 1)
               ).astype(jnp.float32)
        sc0r = _dot(sc0, eye, ((0,), (0,)))
        sc1r = _dot(sc1, eye, ((0,), (0,)))
        s01 = jnp.concatenate([jnp.broadcast_to(sc0r, (8, NB_GRAPH)),
                               jnp.broadcast_to(sc1r, (8, NB_GRAPH))], 0)
        Ahat_ref[...] = (Ahat_ref[...] * s01[:, :, None]
                         + contrib.reshape(16, NB_GRAPH, D))

    @pl.when((ph == 2) & (j == 0))
    def _per_graph():
        A3 = Ahat_ref[...]
        Wqd_g = _dot(Wqg_ref[...], Wdot_ref[...], ((1,), (0,))) / SQRT_D
        mq_blocks = []
        for p in range(2):
            du = jnp.zeros((NB_GRAPH, D), jnp.float32)
            for h in range(HID):
                du = du + _dot(A3[p * HID + h], Wv2_ref[h], ((1,), (0,)))
            inv_col = 1.0 / (S_ref[:, 4 + p:5 + p] + 1e-9)
            nd = jnp.broadcast_to(init_ref[...], (NB_GRAPH, D)) + du * inv_col
            for h in range(HID):
                k2g = _dot(nd, Wkg2_ref[h], ((1,), (0,)))
                mq_blocks.append(_dot(Wqd_g, k2g, ((1,), (1,))))
                V2g_ref[p * HID + h] = _dot(nd, Wvg2_ref[h], ((1,), (0,)))
        Mq_ref[...] = jnp.concatenate(mq_blocks, axis=1)

    @pl.when(ph == 2)
    def _out():
        X = x_ref[...]
        bf_t = BF_ref[j]
        hkg0, hkg1, hvg0, hvg1 = _mixed_feats(bf_t, WM_ref[:, 32:64])
        Ptile = jnp.concatenate([Pf] * 16, axis=0)
        Dm = _dot(Mq_ref[...], X, ((0,), (1,)), _LO)
        T = _dot(R_ref[...], Dm * Ptile, ((1,), (0,)), _LO)
        lg0 = jnp.sum(hkg0 * T[0:HID], axis=0, keepdims=True)
        lg1 = jnp.sum(hkg1 * T[HID:16], axis=0, keepdims=True)
        mx = jnp.maximum(lg0, lg1)
        a0 = jnp.exp(lg0 - mx)
        a1 = jnp.exp(lg1 - mx)
        s = a0 + a1
        coeff_g = jnp.concatenate([(a0 / s) * hvg0, (a1 / s) * hvg1], 0)
        coeffB = _dot(R_ref[...], coeff_g, ((0,), (0,)), _LO) * Ptile
        upd = _dot(coeffB, V2g_ref[...].reshape(16 * NB_GRAPH, D),
                   ((0,), (0,)), _LO)
        out_ref[...] = X + upd


def kernel(pos, node_features, batch, bessel_weights, init_dummy_embedding,
           Wq_dummy, Wk1, Wk2, Wv1, Wv2, Wq_graph, Wkg1, Wkg2, Wvg1, Wvg2,
           Wdot):
    pad = NP - N
    zrow = jnp.pad(pos[:, 2], (0, pad)).reshape(1, NP)
    brow = jnp.pad(batch, (0, pad), constant_values=NB_GRAPH).reshape(1, NP)
    xpad = jnp.pad(node_features, ((0, pad), (0, 0)))
    bwc = bessel_weights.reshape(NBES, 1)
    init2 = init_dummy_embedding.reshape(1, D)
    R = jnp.asarray(_R_CONST)

    row_map = lambda ph, j: (0, j)
    x_map = lambda ph, j: (jnp.where(ph == 0, 0, j), 0)
    out_map = lambda ph, j: (jnp.where(ph == 2, j, 0), 0)
    full2 = lambda ph, j: (0, 0)
    full3 = lambda ph, j: (0, 0, 0)

    in_specs = [
        pl.BlockSpec((1, BLK), row_map),
        pl.BlockSpec((1, BLK), row_map),
        pl.BlockSpec((BLK, D), x_map),
        pl.BlockSpec((NBES, 1), full2),
        pl.BlockSpec((1, D), full2),
        pl.BlockSpec((D, D), full2),
        pl.BlockSpec((D, D), full2),
        pl.BlockSpec((D, D), full2),
        pl.BlockSpec((NBES, HID), full2),
        pl.BlockSpec((NBES, HID), full2),
        pl.BlockSpec((NBES, HID), full2),
        pl.BlockSpec((NBES, HID), full2),
        pl.BlockSpec((HID, D, D), full3),
        pl.BlockSpec((HID, D, D), full3),
        pl.BlockSpec((HID, D, D), full3),
        pl.BlockSpec((HID, D, D), full3),
        pl.BlockSpec((16, 16 * NB_GRAPH), full2),
    ]

    out = pl.pallas_call(
        _body,
        grid=(3, NBLK),
        in_specs=in_specs,
        out_specs=pl.BlockSpec((BLK, D), out_map),
        out_shape=jax.ShapeDtypeStruct((NP, D), jnp.float32),
        scratch_shapes=[
            pltpu.VMEM((NB_GRAPH, 8), jnp.float32),
            pltpu.VMEM((16, NB_GRAPH, D), jnp.float32),
            pltpu.VMEM((D, 16 * NB_GRAPH), jnp.float32),
            pltpu.VMEM((16, NB_GRAPH, D), jnp.float32),
            pltpu.VMEM((D, HID), jnp.float32),
            pltpu.VMEM((16, 64), jnp.float32),
            pltpu.VMEM((NBLK, 16, BLK), jnp.float32),
        ],
    )(zrow, brow, xpad, bwc, init2, Wq_dummy, Wdot, Wq_graph,
      Wk1, Wv1, Wkg1, Wvg1, Wk2, Wv2, Wkg2, Wvg2, R)
    return out[:N]


# speedup vs baseline: 1.3630x; 1.0542x over previous
import jax
import jax.numpy as jnp
import numpy as np
from jax.experimental import pallas as pl
from jax.experimental.pallas import tpu as pltpu

N = 10000
BLK = 5120
NBLK = 2
NP = BLK * NBLK
D = 128
NBES = 8
HID = 8
NB_GRAPH = 64
DIST = 5.0
EPS = 1e-6
NEG = -1e30
SQRT_D = float(np.sqrt(D))

_R_CONST = np.repeat(np.eye(16, dtype=np.float32), NB_GRAPH, axis=1)

_HI = jax.lax.Precision.HIGHEST
_LO = jax.lax.Precision.DEFAULT


def _dot(a, b, dims, prec=_HI):
    return jax.lax.dot_general(a, b, (dims, ((), ())), precision=prec,
                               preferred_element_type=jnp.float32)


def _body(z_ref, b_ref, x_ref, bw_ref, init_ref, Wqd_ref, Wdot_ref, Wqg_ref,
          Wk1_ref, Wv1_ref, Wkg1_ref, Wvg1_ref, Wk2_ref, Wv2_ref, Wkg2_ref,
          Wvg2_ref, R_ref, out_ref, S_ref, Ahat_ref, Mq_ref, V2g_ref, KE_ref,
          WM_ref, BF_ref):
    ph = pl.program_id(0)
    j = pl.program_id(1)

    @pl.when((ph == 0) & (j == 0))
    def _init():
        ci = jax.lax.broadcasted_iota(jnp.int32, (NB_GRAPH, 8), 1)
        S_ref[...] = jnp.where(ci == 0, 1e30,
                               jnp.where(ci < 4, NEG, 0.0)).astype(jnp.float32)
        Ahat_ref[...] = jnp.zeros((16, NB_GRAPH, D), jnp.float32)

    zrow = z_ref[...]
    brow = b_ref[...]
    iota = jax.lax.broadcasted_iota(jnp.int32, (NB_GRAPH, BLK), 0)
    Pb = brow == iota
    Pf = Pb.astype(jnp.float32)

    @pl.when(ph == 0)
    def _zstats():
        zmin = jnp.min(jnp.where(Pb, zrow, 1e30), axis=1, keepdims=True)
        zmax = jnp.max(jnp.where(Pb, zrow, NEG), axis=1, keepdims=True)
        S_ref[:, 0:1] = jnp.minimum(S_ref[:, 0:1], zmin)
        S_ref[:, 1:2] = jnp.maximum(S_ref[:, 1:2], zmax)

    def _mixed_feats(bf_t, wm):
        h = jax.nn.silu(_dot(wm, bf_t, ((0,), (0,))))
        return h[0:8], h[8:16], h[16:24], h[24:32]

    @pl.when((ph == 1) & (j == 0))
    def _shared_weights():
        q0 = _dot(init_ref[...], Wqd_ref[...], ((1,), (0,)))
        wv = _dot(q0, Wdot_ref[...], ((1,), (0,))) / SQRT_D
        KE_ref[...] = jnp.concatenate(
            [_dot(Wk2_ref[h], wv, ((1,), (1,))) for h in range(HID)], axis=1)
        z8 = jnp.zeros((8, 8), jnp.float32)
        blocks = []
        for W in (Wk1_ref[...], Wv1_ref[...], Wkg1_ref[...], Wvg1_ref[...]):
            blocks.append(jnp.concatenate([W, z8], 0))
            blocks.append(jnp.concatenate([z8, W], 0))
        WM_ref[...] = jnp.concatenate(blocks, 1)

    @pl.when(ph == 1)
    def _main():
        X = x_ref[...]
        stats = _dot(S_ref[...], Pf, ((0,), (0,)))
        bottom = zrow - stats[0:1] + DIST
        top = stats[1:2] + DIST - zrow
        bw = bw_ref[...]
        bfin = jnp.concatenate([bw * bottom, bw * top], 0)
        denv = jnp.concatenate(
            [jnp.broadcast_to(bottom + EPS, (NBES, BLK)),
             jnp.broadcast_to(top + EPS, (NBES, BLK))], 0)
        bf_t = jnp.sin(bfin) / denv
        BF_ref[j] = bf_t
        hk0, hk1, hv0, hv1 = _mixed_feats(bf_t, WM_ref[:, 0:32])
        s_k = _dot(KE_ref[...], X, ((0,), (1,)))
        l0 = jnp.sum(hk0 * s_k, axis=0, keepdims=True)
        l1 = jnp.sum(hk1 * s_k, axis=0, keepdims=True)
        bm0 = jnp.max(jnp.where(Pb, l0, NEG), axis=1, keepdims=True)
        bm1 = jnp.max(jnp.where(Pb, l1, NEG), axis=1, keepdims=True)
        old0 = S_ref[:, 2:3]
        old1 = S_ref[:, 3:4]
        new0 = jnp.maximum(old0, bm0)
        new1 = jnp.maximum(old1, bm1)
        sc0 = jnp.exp(old0 - new0)
        sc1 = jnp.exp(old1 - new1)
        S_ref[:, 2:3] = new0
        S_ref[:, 3:4] = new1
        m_n = _dot(jnp.concatenate([new0, new1], 1), Pf, ((0,), (0,)))
        e0 = jnp.exp(l0 - m_n[0:1])
        e1 = jnp.exp(l1 - m_n[1:2])
        sum_e = _dot(Pf, jnp.concatenate([e0, e1], 0), ((1,), (1,)))
        S_ref[:, 4:6] = (S_ref[:, 4:6] * jnp.concatenate([sc0, sc1], 1)
                         + sum_e)
        coeff = jnp.concatenate([e0 * hv0, e1 * hv1], 0)
        coeff_rep = _dot(R_ref[...], coeff, ((0,), (0,)), _LO)
        Ptile = jnp.concatenate([Pf] * 16, axis=0)
        contrib = _dot(coeff_rep * Ptile, X, ((1,), (0,)), _LO)
        eye = (jax.lax.broadcasted_iota(jnp.int32, (NB_GRAPH, NB_GRAPH), 0)
               == jax.lax.broadcasted_iota(jnp.int32, (NB_GRAPH, NB_GRAPH), 1)
               ).astype(jnp.float32)
        sc0r = _dot(sc0, eye, ((0,), (0,)))
        sc1r = _dot(sc1, eye, ((0,), (0,)))
        s01 = jnp.concatenate([jnp.broadcast_to(sc0r, (8, NB_GRAPH)),
                               jnp.broadcast_to(sc1r, (8, NB_GRAPH))], 0)
        Ahat_ref[...] = (Ahat_ref[...] * s01[:, :, None]
                         + contrib.reshape(16, NB_GRAPH, D))

    @pl.when((ph == 2) & (j == 0))
    def _per_graph():
        A3 = Ahat_ref[...]
        Wqd_g = _dot(Wqg_ref[...], Wdot_ref[...], ((1,), (0,))) / SQRT_D
        mq_blocks = []
        for p in range(2):
            du = jnp.zeros((NB_GRAPH, D), jnp.float32)
            for h in range(HID):
                du = du + _dot(A3[p * HID + h], Wv2_ref[h], ((1,), (0,)))
            inv_col = 1.0 / (S_ref[:, 4 + p:5 + p] + 1e-9)
            nd = jnp.broadcast_to(init_ref[...], (NB_GRAPH, D)) + du * inv_col
            for h in range(HID):
                k2g = _dot(nd, Wkg2_ref[h], ((1,), (0,)))
                mq_blocks.append(_dot(Wqd_g, k2g, ((1,), (1,))))
                V2g_ref[p * HID + h] = _dot(nd, Wvg2_ref[h], ((1,), (0,)))
        Mq_ref[...] = jnp.concatenate(mq_blocks, axis=1)

    @pl.when(ph == 2)
    def _out():
        X = x_ref[...]
        bf_t = BF_ref[j]
        hkg0, hkg1, hvg0, hvg1 = _mixed_feats(bf_t, WM_ref[:, 32:64])
        Ptile = jnp.concatenate([Pf] * 16, axis=0)
        Dm = _dot(Mq_ref[...], X, ((0,), (1,)), _LO)
        T = _dot(R_ref[...], Dm * Ptile, ((1,), (0,)), _LO)
        lg0 = jnp.sum(hkg0 * T[0:HID], axis=0, keepdims=True)
        lg1 = jnp.sum(hkg1 * T[HID:16], axis=0, keepdims=True)
        mx = jnp.maximum(lg0, lg1)
        a0 = jnp.exp(lg0 - mx)
        a1 = jnp.exp(lg1 - mx)
        s = a0 + a1
        coeff_g = jnp.concatenate([(a0 / s) * hvg0, (a1 / s) * hvg1], 0)
        coeffB = _dot(R_ref[...], coeff_g, ((0,), (0,)), _LO) * Ptile
        upd = _dot(coeffB, V2g_ref[...].reshape(16 * NB_GRAPH, D),
                   ((0,), (0,)), _LO)
        out_ref[...] = X + upd


def kernel(pos, node_features, batch, bessel_weights, init_dummy_embedding,
           Wq_dummy, Wk1, Wk2, Wv1, Wv2, Wq_graph, Wkg1, Wkg2, Wvg1, Wvg2,
           Wdot):
    pad = NP - N
    zrow = jnp.pad(pos[:, 2], (0, pad)).reshape(1, NP)
    brow = jnp.pad(batch, (0, pad), constant_values=NB_GRAPH).reshape(1, NP)
    xpad = jnp.pad(node_features, ((0, pad), (0, 0)))
    bwc = bessel_weights.reshape(NBES, 1)
    init2 = init_dummy_embedding.reshape(1, D)
    R = jnp.asarray(_R_CONST)

    row_map = lambda ph, j: (0, j)
    x_map = lambda ph, j: (jnp.where(ph == 0, 0, j), 0)
    out_map = lambda ph, j: (jnp.where(ph == 2, j, 0), 0)
    full2 = lambda ph, j: (0, 0)
    full3 = lambda ph, j: (0, 0, 0)

    in_specs = [
        pl.BlockSpec((1, BLK), row_map),
        pl.BlockSpec((1, BLK), row_map),
        pl.BlockSpec((BLK, D), x_map),
        pl.BlockSpec((NBES, 1), full2),
        pl.BlockSpec((1, D), full2),
        pl.BlockSpec((D, D), full2),
        pl.BlockSpec((D, D), full2),
        pl.BlockSpec((D, D), full2),
        pl.BlockSpec((NBES, HID), full2),
        pl.BlockSpec((NBES, HID), full2),
        pl.BlockSpec((NBES, HID), full2),
        pl.BlockSpec((NBES, HID), full2),
        pl.BlockSpec((HID, D, D), full3),
        pl.BlockSpec((HID, D, D), full3),
        pl.BlockSpec((HID, D, D), full3),
        pl.BlockSpec((HID, D, D), full3),
        pl.BlockSpec((16, 16 * NB_GRAPH), full2),
    ]

    out = pl.pallas_call(
        _body,
        grid=(3, NBLK),
        in_specs=in_specs,
        out_specs=pl.BlockSpec((BLK, D), out_map),
        out_shape=jax.ShapeDtypeStruct((NP, D), jnp.float32),
        scratch_shapes=[
            pltpu.VMEM((NB_GRAPH, 8), jnp.float32),
            pltpu.VMEM((16, NB_GRAPH, D), jnp.float32),
            pltpu.VMEM((D, 16 * NB_GRAPH), jnp.float32),
            pltpu.VMEM((16, NB_GRAPH, D), jnp.float32),
            pltpu.VMEM((D, HID), jnp.float32),
            pltpu.VMEM((16, 64), jnp.float32),
            pltpu.VMEM((NBLK, 16, BLK), jnp.float32),
        ],
    )(zrow, brow, xpad, bwc, init2, Wq_dummy, Wdot, Wq_graph,
      Wk1, Wv1, Wkg1, Wvg1, Wk2, Wv2, Wkg2, Wvg2, R)
    return out[:N]
